# Initial kernel scaffold; baseline (speedup 1.0000x reference)
import jax
import jax.numpy as jnp
from jax.experimental import pallas as pl


def kernel(feat, depth, edge_index, W_enc, b_enc, depth_emb, W1, b1, W2, b2, W3, b3, gamma, beta):
    raise NotImplementedError("write your pallas kernel here")


# speedup vs baseline: 4.3634x; 4.3634x over previous
import functools

import jax
import jax.numpy as jnp
from jax import lax
from jax.experimental import pallas as pl
from jax.experimental.pallas import tpu as pltpu
from jax.experimental.pallas import tpu_sc as plsc

_N = 10000
_E = 320000
_D = 128
_L = 3
_VOCAB = 32

_NC = 2
_NS = 16
_NW = _NC * _NS
_EPW = _E // _NW
_CHUNK = 80
_NCHUNK = _EPW // _CHUNK
_RPT = 632

_BLK = 2000
_NBLK = _N // _BLK


def _sc_agg_body(x_hbm, src_hbm, dst_hbm, zeros_hbm, out_hbm,
                 idx_s, idx_d, rows, aggsh, sem):
    c = lax.axis_index("c")
    s = lax.axis_index("s")
    w = c * _NS + s

    stripe = pl.multiple_of(
        jnp.minimum(s * _RPT, _N - _RPT).astype(jnp.int32), 8)
    pltpu.sync_copy(zeros_hbm.at[pl.ds(stripe, _RPT)],
                    aggsh.at[pl.ds(stripe, _RPT)])
    plsc.subcore_barrier()

    ebase = w * _EPW

    def body(g, carry):
        base = ebase + g * _CHUNK
        pltpu.sync_copy(src_hbm.at[pl.ds(base, _CHUNK)], idx_s)
        pltpu.sync_copy(dst_hbm.at[pl.ds(base, _CHUNK)], idx_d)
        pltpu.async_copy(x_hbm.at[idx_s], rows, sem).wait()
        pltpu.sync_copy(rows, aggsh.at[idx_d], add=True)
        return carry

    lax.fori_loop(0, _NCHUNK, body, 0)
    plsc.subcore_barrier()

    pltpu.sync_copy(aggsh.at[pl.ds(stripe, _RPT)],
                    out_hbm.at[c, pl.ds(stripe, _RPT)])


@functools.lru_cache(maxsize=None)
def _sc_agg_kernel():
    return functools.partial(
        pl.kernel,
        out_type=jax.ShapeDtypeStruct((_NC, _N, _D), jnp.float32),
        mesh=plsc.VectorSubcoreMesh(core_axis_name="c", subcore_axis_name="s"),
        scratch_types=[
            pltpu.VMEM((_CHUNK,), jnp.int32),
            pltpu.VMEM((_CHUNK,), jnp.int32),
            pltpu.VMEM((_CHUNK, _D), jnp.float32),
            pltpu.VMEM_SHARED((_N, _D), jnp.float32),
            pltpu.SemaphoreType.DMA,
        ],
    )(_sc_agg_body)


def _sc_agg(x, src, dst, zeros):
    return _sc_agg_kernel()(x, src, dst, zeros)


_HI = lax.Precision.HIGHEST


def _enc_body(feat_ref, depth_ref, wenc_ref, benc_ref, demb_ref, out_ref):
    x = jnp.dot(feat_ref[...], wenc_ref[...],
                preferred_element_type=jnp.float32)
    oh = (depth_ref[...] == lax.broadcasted_iota(
        jnp.int32, (_BLK, _VOCAB), 1)).astype(jnp.float32)
    emb = jnp.dot(oh, demb_ref[...],
                  preferred_element_type=jnp.float32, precision=_HI)
    out_ref[...] = x + benc_ref[...] + emb


def _mlp_body(x_ref, a0_ref, a1_ref, w1, b1, w2, b2, w3, b3,
              h_ref, stats_ref):
    i = pl.program_id(0)
    t = x_ref[...] + a0_ref[...] + a1_ref[...]
    h = jnp.maximum(jnp.dot(t, w1[...],
                            preferred_element_type=jnp.float32) + b1[...], 0.0)
    h = jnp.maximum(jnp.dot(h, w2[...],
                            preferred_element_type=jnp.float32) + b2[...], 0.0)
    h = jnp.maximum(jnp.dot(h, w3[...],
                            preferred_element_type=jnp.float32) + b3[...], 0.0)
    h_ref[...] = h
    bmean = jnp.mean(h, axis=0, keepdims=True)
    bm2 = jnp.sum((h - bmean) * (h - bmean), axis=0, keepdims=True)

    @pl.when(i == 0)
    def _():
        stats_ref[...] = jnp.concatenate([bmean, bm2], axis=0)

    @pl.when(i != 0)
    def _():
        n = i * _BLK * 1.0
        mean = stats_ref[0:1]
        m2 = stats_ref[1:2]
        delta = bmean - mean
        tot = n + _BLK
        new_mean = mean + delta * (_BLK / tot)
        new_m2 = m2 + bm2 + delta * delta * (n * _BLK / tot)
        stats_ref[...] = jnp.concatenate([new_mean, new_m2], axis=0)


def _bn_body(h_ref, stats_ref, g_ref, b_ref, out_ref, *, apply_relu):
    mean = stats_ref[0:1]
    var = stats_ref[1:2] * (1.0 / _N)
    inv = g_ref[...] * lax.rsqrt(var + 1e-5)
    o = (h_ref[...] - mean) * inv + b_ref[...]
    if apply_relu:
        o = jnp.maximum(o, 0.0)
    out_ref[...] = o


_row_spec = pl.BlockSpec((_BLK, _D), lambda i: (i, 0))
_full_spec = pl.BlockSpec(lambda i: (0, 0))


def _enc_call(feat, depth2d, wenc, benc2d, demb):
    return pl.pallas_call(
        _enc_body,
        grid=(_NBLK,),
        in_specs=[
            _row_spec,
            pl.BlockSpec((_BLK, 1), lambda i: (i, 0)),
            pl.BlockSpec((_D, _D), lambda i: (0, 0)),
            pl.BlockSpec((1, _D), lambda i: (0, 0)),
            pl.BlockSpec((_VOCAB, _D), lambda i: (0, 0)),
        ],
        out_specs=_row_spec,
        out_shape=jax.ShapeDtypeStruct((_N, _D), jnp.float32),
    )(feat, depth2d, wenc, benc2d, demb)


def _mlp_call(x, a0, a1, w1, b1, w2, b2, w3, b3):
    wspec = pl.BlockSpec((_D, _D), lambda i: (0, 0))
    bspec = pl.BlockSpec((1, _D), lambda i: (0, 0))
    return pl.pallas_call(
        _mlp_body,
        grid=(_NBLK,),
        in_specs=[_row_spec, _row_spec, _row_spec,
                  wspec, bspec, wspec, bspec, wspec, bspec],
        out_specs=[_row_spec, pl.BlockSpec((2, _D), lambda i: (0, 0))],
        out_shape=[jax.ShapeDtypeStruct((_N, _D), jnp.float32),
                   jax.ShapeDtypeStruct((2, _D), jnp.float32)],
    )(x, a0, a1, w1, b1, w2, b2, w3, b3)


def _bn_call(h, stats, g, b, apply_relu):
    return pl.pallas_call(
        functools.partial(_bn_body, apply_relu=apply_relu),
        grid=(_NBLK,),
        in_specs=[_row_spec,
                  pl.BlockSpec((2, _D), lambda i: (0, 0)),
                  pl.BlockSpec((1, _D), lambda i: (0, 0)),
                  pl.BlockSpec((1, _D), lambda i: (0, 0))],
        out_specs=_row_spec,
        out_shape=jax.ShapeDtypeStruct((_N, _D), jnp.float32),
    )(h, stats, g, b)


def kernel(feat, depth, edge_index, W_enc, b_enc, depth_emb,
           W1, b1, W2, b2, W3, b3, gamma, beta):
    src = edge_index[0]
    dst = edge_index[1]
    zeros = jnp.zeros((_N, _D), jnp.float32)

    x = _enc_call(feat, depth.reshape(_N, 1), W_enc,
                  b_enc.reshape(1, _D), depth_emb)

    b1r = b1.reshape(1, _D)
    b2r = b2.reshape(1, _D)
    b3r = b3.reshape(1, _D)
    for l in range(_L):
        agg = _sc_agg(x, src, dst, zeros)
        h, stats = _mlp_call(x, agg[0], agg[1], W1, b1r, W2, b2r, W3, b3r)
        x = _bn_call(h, stats, gamma[l].reshape(1, _D),
                     beta[l].reshape(1, _D), apply_relu=(l < _L - 1))
    return x
